# baseline (speedup 1.0000x reference)
import jax
import jax.numpy as jnp
from jax.experimental import pallas as pl
from jax.experimental.pallas import tpu as pltpu

_P = 512
_R = 0.05
_TI = 32
_F = 32
_C = 4
_L = _F * _C
_Q = _P // _C


def _grad_body(xi_ref, xj4_ref, Wa4_ref, Wb16_ref, sjx_ref, sjy_ref,
               b1_4_ref, w1d4_ref, W2b_ref, W2bT_ref, b2_4_ref, wo4_ref,
               VBS_ref, V1X_ref, V1Y_ref, V2X_ref, V2Y_ref, RED4_ref,
               gi_ref, gj_ref):
    it = pl.program_id(1)
    xi = xi_ref[0]
    xj4 = xj4_ref[0]

    A4 = jnp.dot(xi, Wa4_ref[...], preferred_element_type=jnp.float32) + b1_4_ref[...]
    B4 = jnp.dot(xj4, Wb16_ref[...], preferred_element_type=jnp.float32)

    pix = jnp.broadcast_to(xi[:, 0:1], (_TI, _L))
    piy = jnp.broadcast_to(xi[:, 1:2], (_TI, _L))
    pjx = jnp.dot(xj4, sjx_ref[...], preferred_element_type=jnp.float32)
    pjy = jnp.dot(xj4, sjy_ref[...], preferred_element_type=jnp.float32)

    relx = pix[:, None, :] - pjx[None, :, :]
    rely = piy[:, None, :] - pjy[None, :, :]
    dist2 = ((pix * pix + piy * piy)[:, None, :]
             + (pjx * pjx + pjy * pjy)[None, :, :]
             - 2.0 * (pix[:, None, :] * pjx[None, :, :]
                      + piy[:, None, :] * pjy[None, :, :]))
    j_id = (4 * jax.lax.broadcasted_iota(jnp.int32, (_Q, _L), 0)
            + jax.lax.broadcasted_iota(jnp.int32, (_Q, _L), 1) // _F)
    i_id = it * _TI + jax.lax.broadcasted_iota(jnp.int32, (_TI, _Q, _L), 0)
    mask = (dist2 < _R * _R) & (i_id != j_id[None, :, :])
    r2 = relx * relx + rely * rely + 1e-8
    rdist = jax.lax.rsqrt(r2)
    dist = r2 * rdist

    z1 = A4[:, None, :] + B4[None, :, :] + dist * w1d4_ref[...][0][None, None, :]
    h = jnp.tanh(z1).reshape(_TI * _Q, _L)
    z2 = jnp.dot(h, W2b_ref[...], preferred_element_type=jnp.float32) + b2_4_ref[...]
    t2 = jnp.tanh(z2)
    maskf = mask.reshape(_TI * _Q, _L)
    dz2 = jnp.where(maskf, (1.0 - t2 * t2) * wo4_ref[...], 0.0)
    dh = jnp.dot(dz2, W2bT_ref[...], preferred_element_type=jnp.float32)
    dz1 = dh * (1.0 - h * h)

    def red(v_ref):
        r = jnp.dot(dz1, v_ref[...], preferred_element_type=jnp.float32)
        return r.reshape(_TI, _Q, _L)

    srd = red(VBS_ref) * rdist
    sux = srd * relx
    suy = srd * rely
    v1x = red(V1X_ref) + sux
    v1y = red(V1Y_ref) + suy
    v2x = red(V2X_ref) - sux
    v2y = red(V2Y_ref) - suy

    gi_x = jnp.sum(v1x, axis=(1, 2)) * (1.0 / _F)
    gi_y = jnp.sum(v1y, axis=(1, 2)) * (1.0 / _F)
    gj2x = jnp.sum(v2x, axis=0)
    gj2y = jnp.sum(v2y, axis=0)
    RED4 = RED4_ref[...]
    gj4 = jnp.concatenate(
        [jnp.dot(gj2x, RED4, preferred_element_type=jnp.float32),
         jnp.dot(gj2y, RED4, preferred_element_type=jnp.float32)], axis=1)

    gi_ref[0, 0] = jnp.stack([gi_x, gi_y], axis=0)

    @pl.when(it == 0)
    def _():
        gj_ref[...] = jnp.zeros_like(gj_ref)

    gj_ref[0] = gj_ref[0] + gj4


def _grad_step(xr, xr4, consts):
    nb = xr.shape[0]
    grid = (nb, _P // _TI)

    def wspec(a):
        return pl.BlockSpec(a.shape, lambda b, it: (0,) * a.ndim)

    gi, gj = pl.pallas_call(
        _grad_body,
        grid=grid,
        in_specs=[
            pl.BlockSpec((1, _TI, 4), lambda b, it: (b, it, 0)),
            pl.BlockSpec((1, _Q, 16), lambda b, it: (b, 0, 0)),
        ] + [wspec(c) for c in consts],
        out_specs=[
            pl.BlockSpec((1, 1, 2, _TI), lambda b, it: (b, it, 0, 0)),
            pl.BlockSpec((1, _Q, 8), lambda b, it: (b, 0, 0)),
        ],
        out_shape=[
            jax.ShapeDtypeStruct((nb, _P // _TI, 2, _TI), jnp.float32),
            jax.ShapeDtypeStruct((nb, _Q, 8), jnp.float32),
        ],
        compiler_params=pltpu.CompilerParams(
            dimension_semantics=("parallel", "arbitrary")),
    )(xr, xr4, *consts)
    return gi, gj


def kernel(x, batch, steps, W1, b1, W2, b2, Wout, bout):
    N = x.shape[0]
    nb = N // _P
    f32 = jnp.float32

    Wr = W1[8:10]
    pad = jnp.zeros((2, _F), dtype=f32)
    Wa = W1[0:4] + jnp.concatenate([Wr, pad], axis=0)
    Wb = W1[4:8] - jnp.concatenate([Wr, pad], axis=0)
    w1d = W1[10:11]
    c1x = W1[0] + W1[8]
    c1y = W1[1] + W1[9]
    c2x = W1[4] - W1[8]
    c2y = W1[5] - W1[9]

    eye4 = jnp.eye(_C, dtype=f32)
    ones1F = jnp.ones((1, _F), dtype=f32)

    def bcmat(vec):
        return jnp.kron(eye4, vec[:, None] @ ones1F)

    e0 = jnp.zeros((4, 1), dtype=f32).at[0, 0].set(1.0)
    e1 = jnp.zeros((4, 1), dtype=f32).at[1, 0].set(1.0)

    consts = (
        jnp.tile(Wa, (1, _C)),
        jnp.kron(eye4, Wb),
        jnp.kron(eye4, e0 @ ones1F),
        jnp.kron(eye4, e1 @ ones1F),
        jnp.tile(b1[None, :], (1, _C)),
        jnp.tile(w1d, (1, _C)),
        jnp.kron(eye4, W2),
        jnp.kron(eye4, W2.T),
        jnp.tile(b2[None, :], (1, _C)),
        jnp.tile(Wout[:, 0][None, :], (1, _C)),
        bcmat(w1d[0]),
        bcmat(c1x), bcmat(c1y), bcmat(c2x), bcmat(c2y),
        jnp.kron(eye4, jnp.ones((_F, 1), dtype=f32) / _F),
    )

    def body(_, xc):
        xr = xc.reshape(nb, _P, 4)
        xr4 = xc.reshape(nb, _Q, 16)
        gi, gj = _grad_step(xr, xr4, consts)
        gix = jnp.transpose(gi, (0, 2, 1, 3)).reshape(nb, 2, _P)
        gjx = gj[:, :, 0:4].reshape(nb, _P)
        gjy = gj[:, :, 4:8].reshape(nb, _P)
        gx = (gix[:, 0] + gjx).reshape(N)
        gy = (gix[:, 1] + gjy).reshape(N)
        newx = xc[:, 0:2] - 0.01 * jnp.stack([gx, gy], axis=1)
        return jnp.concatenate([newx, xc[:, 2:]], axis=1)

    return jax.lax.fori_loop(0, steps, body, x)


# speedup vs baseline: 1.0662x; 1.0662x over previous
import jax
import jax.numpy as jnp
from jax.experimental import pallas as pl
from jax.experimental.pallas import tpu as pltpu

_P = 512
_R = 0.05
_TI = 64
_F = 32
_C = 4
_L = _F * _C
_Q = _P // _C


def _grad_body(xi_ref, xj4_ref, Wa4_ref, Wb16_ref, sjx_ref, sjy_ref,
               b1_4_ref, w1d4_ref, W2b_ref, W2bT_ref, b2_4_ref, wo4_ref,
               VBS_ref, V1X_ref, V1Y_ref, V2X_ref, V2Y_ref, RED4_ref,
               gi_ref, gj_ref):
    it = pl.program_id(1)
    xi = xi_ref[0]
    xj4 = xj4_ref[0]

    A4 = jnp.dot(xi, Wa4_ref[...], preferred_element_type=jnp.float32) + b1_4_ref[...]
    B4 = jnp.dot(xj4, Wb16_ref[...], preferred_element_type=jnp.float32)

    pix = jnp.broadcast_to(xi[:, 0:1], (_TI, _L))
    piy = jnp.broadcast_to(xi[:, 1:2], (_TI, _L))
    pjx = jnp.dot(xj4, sjx_ref[...], preferred_element_type=jnp.float32)
    pjy = jnp.dot(xj4, sjy_ref[...], preferred_element_type=jnp.float32)

    relx = pix[:, None, :] - pjx[None, :, :]
    rely = piy[:, None, :] - pjy[None, :, :]
    dist2 = ((pix * pix + piy * piy)[:, None, :]
             + (pjx * pjx + pjy * pjy)[None, :, :]
             - 2.0 * (pix[:, None, :] * pjx[None, :, :]
                      + piy[:, None, :] * pjy[None, :, :]))
    j_id = (4 * jax.lax.broadcasted_iota(jnp.int32, (_Q, _L), 0)
            + jax.lax.broadcasted_iota(jnp.int32, (_Q, _L), 1) // _F)
    i_id = it * _TI + jax.lax.broadcasted_iota(jnp.int32, (_TI, _Q, _L), 0)
    mask = (dist2 < _R * _R) & (i_id != j_id[None, :, :])
    r2 = relx * relx + rely * rely + 1e-8
    rdist = jax.lax.rsqrt(r2)
    dist = r2 * rdist

    z1 = A4[:, None, :] + B4[None, :, :] + dist * w1d4_ref[...][0][None, None, :]
    h = jnp.tanh(z1).reshape(_TI * _Q, _L)
    z2 = jnp.dot(h, W2b_ref[...], preferred_element_type=jnp.float32) + b2_4_ref[...]
    t2 = jnp.tanh(z2)
    maskf = mask.reshape(_TI * _Q, _L)
    dz2 = jnp.where(maskf, (1.0 - t2 * t2) * wo4_ref[...], 0.0)
    dh = jnp.dot(dz2, W2bT_ref[...], preferred_element_type=jnp.float32)
    dz1 = dh * (1.0 - h * h)

    def red(v_ref):
        r = jnp.dot(dz1, v_ref[...], preferred_element_type=jnp.float32)
        return r.reshape(_TI, _Q, _L)

    srd = red(VBS_ref) * rdist
    sux = srd * relx
    suy = srd * rely
    v1x = red(V1X_ref) + sux
    v1y = red(V1Y_ref) + suy
    v2x = red(V2X_ref) - sux
    v2y = red(V2Y_ref) - suy

    gi_x = jnp.sum(v1x, axis=(1, 2)) * (1.0 / _F)
    gi_y = jnp.sum(v1y, axis=(1, 2)) * (1.0 / _F)
    gj2x = jnp.sum(v2x, axis=0)
    gj2y = jnp.sum(v2y, axis=0)
    RED4 = RED4_ref[...]
    gj4 = jnp.concatenate(
        [jnp.dot(gj2x, RED4, preferred_element_type=jnp.float32),
         jnp.dot(gj2y, RED4, preferred_element_type=jnp.float32)], axis=1)

    gi_ref[0, 0] = jnp.stack([gi_x, gi_y], axis=0)

    @pl.when(it == 0)
    def _():
        gj_ref[...] = jnp.zeros_like(gj_ref)

    gj_ref[0] = gj_ref[0] + gj4


def _grad_step(xr, xr4, consts):
    nb = xr.shape[0]
    grid = (nb, _P // _TI)

    def wspec(a):
        return pl.BlockSpec(a.shape, lambda b, it: (0,) * a.ndim)

    gi, gj = pl.pallas_call(
        _grad_body,
        grid=grid,
        in_specs=[
            pl.BlockSpec((1, _TI, 4), lambda b, it: (b, it, 0)),
            pl.BlockSpec((1, _Q, 16), lambda b, it: (b, 0, 0)),
        ] + [wspec(c) for c in consts],
        out_specs=[
            pl.BlockSpec((1, 1, 2, _TI), lambda b, it: (b, it, 0, 0)),
            pl.BlockSpec((1, _Q, 8), lambda b, it: (b, 0, 0)),
        ],
        out_shape=[
            jax.ShapeDtypeStruct((nb, _P // _TI, 2, _TI), jnp.float32),
            jax.ShapeDtypeStruct((nb, _Q, 8), jnp.float32),
        ],
        compiler_params=pltpu.CompilerParams(
            dimension_semantics=("parallel", "arbitrary")),
    )(xr, xr4, *consts)
    return gi, gj


def kernel(x, batch, steps, W1, b1, W2, b2, Wout, bout):
    N = x.shape[0]
    nb = N // _P
    f32 = jnp.float32

    Wr = W1[8:10]
    pad = jnp.zeros((2, _F), dtype=f32)
    Wa = W1[0:4] + jnp.concatenate([Wr, pad], axis=0)
    Wb = W1[4:8] - jnp.concatenate([Wr, pad], axis=0)
    w1d = W1[10:11]
    c1x = W1[0] + W1[8]
    c1y = W1[1] + W1[9]
    c2x = W1[4] - W1[8]
    c2y = W1[5] - W1[9]

    eye4 = jnp.eye(_C, dtype=f32)
    ones1F = jnp.ones((1, _F), dtype=f32)

    def bcmat(vec):
        return jnp.kron(eye4, vec[:, None] @ ones1F)

    e0 = jnp.zeros((4, 1), dtype=f32).at[0, 0].set(1.0)
    e1 = jnp.zeros((4, 1), dtype=f32).at[1, 0].set(1.0)

    consts = (
        jnp.tile(Wa, (1, _C)),
        jnp.kron(eye4, Wb),
        jnp.kron(eye4, e0 @ ones1F),
        jnp.kron(eye4, e1 @ ones1F),
        jnp.tile(b1[None, :], (1, _C)),
        jnp.tile(w1d, (1, _C)),
        jnp.kron(eye4, W2),
        jnp.kron(eye4, W2.T),
        jnp.tile(b2[None, :], (1, _C)),
        jnp.tile(Wout[:, 0][None, :], (1, _C)),
        bcmat(w1d[0]),
        bcmat(c1x), bcmat(c1y), bcmat(c2x), bcmat(c2y),
        jnp.kron(eye4, jnp.ones((_F, 1), dtype=f32) / _F),
    )

    def body(_, xc):
        xr = xc.reshape(nb, _P, 4)
        xr4 = xc.reshape(nb, _Q, 16)
        gi, gj = _grad_step(xr, xr4, consts)
        gix = jnp.transpose(gi, (0, 2, 1, 3)).reshape(nb, 2, _P)
        gjx = gj[:, :, 0:4].reshape(nb, _P)
        gjy = gj[:, :, 4:8].reshape(nb, _P)
        gx = (gix[:, 0] + gjx).reshape(N)
        gy = (gix[:, 1] + gjy).reshape(N)
        newx = xc[:, 0:2] - 0.01 * jnp.stack([gx, gy], axis=1)
        return jnp.concatenate([newx, xc[:, 2:]], axis=1)

    return jax.lax.fori_loop(0, steps, body, x)


# speedup vs baseline: 1.1086x; 1.0398x over previous
import jax
import jax.numpy as jnp
from jax.experimental import pallas as pl
from jax.experimental.pallas import tpu as pltpu

_P = 512
_R = 0.05
_TI = 128
_F = 32
_C = 4
_L = _F * _C
_Q = _P // _C


def _grad_body(xi_ref, xj4_ref, Wa4_ref, Wb16_ref, sjx_ref, sjy_ref,
               b1_4_ref, w1d4_ref, W2b_ref, W2bT_ref, b2_4_ref, wo4_ref,
               VBS_ref, V1X_ref, V1Y_ref, V2X_ref, V2Y_ref, RED4_ref,
               gi_ref, gj_ref):
    it = pl.program_id(1)
    xi = xi_ref[0]
    xj4 = xj4_ref[0]

    A4 = jnp.dot(xi, Wa4_ref[...], preferred_element_type=jnp.float32) + b1_4_ref[...]
    B4 = jnp.dot(xj4, Wb16_ref[...], preferred_element_type=jnp.float32)

    pix = jnp.broadcast_to(xi[:, 0:1], (_TI, _L))
    piy = jnp.broadcast_to(xi[:, 1:2], (_TI, _L))
    pjx = jnp.dot(xj4, sjx_ref[...], preferred_element_type=jnp.float32)
    pjy = jnp.dot(xj4, sjy_ref[...], preferred_element_type=jnp.float32)

    relx = pix[:, None, :] - pjx[None, :, :]
    rely = piy[:, None, :] - pjy[None, :, :]
    dist2 = ((pix * pix + piy * piy)[:, None, :]
             + (pjx * pjx + pjy * pjy)[None, :, :]
             - 2.0 * (pix[:, None, :] * pjx[None, :, :]
                      + piy[:, None, :] * pjy[None, :, :]))
    j_id = (4 * jax.lax.broadcasted_iota(jnp.int32, (_Q, _L), 0)
            + jax.lax.broadcasted_iota(jnp.int32, (_Q, _L), 1) // _F)
    i_id = it * _TI + jax.lax.broadcasted_iota(jnp.int32, (_TI, _Q, _L), 0)
    mask = (dist2 < _R * _R) & (i_id != j_id[None, :, :])
    r2 = relx * relx + rely * rely + 1e-8
    rdist = jax.lax.rsqrt(r2)
    dist = r2 * rdist

    z1 = A4[:, None, :] + B4[None, :, :] + dist * w1d4_ref[...][0][None, None, :]
    h = jnp.tanh(z1).reshape(_TI * _Q, _L)
    z2 = jnp.dot(h, W2b_ref[...], preferred_element_type=jnp.float32) + b2_4_ref[...]
    t2 = jnp.tanh(z2)
    maskf = mask.reshape(_TI * _Q, _L)
    dz2 = jnp.where(maskf, (1.0 - t2 * t2) * wo4_ref[...], 0.0)
    dh = jnp.dot(dz2, W2bT_ref[...], preferred_element_type=jnp.float32)
    dz1 = dh * (1.0 - h * h)

    def red(v_ref):
        r = jnp.dot(dz1, v_ref[...], preferred_element_type=jnp.float32)
        return r.reshape(_TI, _Q, _L)

    srd = red(VBS_ref) * rdist
    sux = srd * relx
    suy = srd * rely
    v1x = red(V1X_ref) + sux
    v1y = red(V1Y_ref) + suy
    v2x = red(V2X_ref) - sux
    v2y = red(V2Y_ref) - suy

    gi_x = jnp.sum(v1x, axis=(1, 2)) * (1.0 / _F)
    gi_y = jnp.sum(v1y, axis=(1, 2)) * (1.0 / _F)
    gj2x = jnp.sum(v2x, axis=0)
    gj2y = jnp.sum(v2y, axis=0)
    RED4 = RED4_ref[...]
    gj4 = jnp.concatenate(
        [jnp.dot(gj2x, RED4, preferred_element_type=jnp.float32),
         jnp.dot(gj2y, RED4, preferred_element_type=jnp.float32)], axis=1)

    gi_ref[0, 0] = jnp.stack([gi_x, gi_y], axis=0)

    @pl.when(it == 0)
    def _():
        gj_ref[...] = jnp.zeros_like(gj_ref)

    gj_ref[0] = gj_ref[0] + gj4


def _grad_step(xr, xr4, consts):
    nb = xr.shape[0]
    grid = (nb, _P // _TI)

    def wspec(a):
        return pl.BlockSpec(a.shape, lambda b, it: (0,) * a.ndim)

    gi, gj = pl.pallas_call(
        _grad_body,
        grid=grid,
        in_specs=[
            pl.BlockSpec((1, _TI, 4), lambda b, it: (b, it, 0)),
            pl.BlockSpec((1, _Q, 16), lambda b, it: (b, 0, 0)),
        ] + [wspec(c) for c in consts],
        out_specs=[
            pl.BlockSpec((1, 1, 2, _TI), lambda b, it: (b, it, 0, 0)),
            pl.BlockSpec((1, _Q, 8), lambda b, it: (b, 0, 0)),
        ],
        out_shape=[
            jax.ShapeDtypeStruct((nb, _P // _TI, 2, _TI), jnp.float32),
            jax.ShapeDtypeStruct((nb, _Q, 8), jnp.float32),
        ],
        compiler_params=pltpu.CompilerParams(
            dimension_semantics=("parallel", "arbitrary")),
    )(xr, xr4, *consts)
    return gi, gj


def kernel(x, batch, steps, W1, b1, W2, b2, Wout, bout):
    N = x.shape[0]
    nb = N // _P
    f32 = jnp.float32

    Wr = W1[8:10]
    pad = jnp.zeros((2, _F), dtype=f32)
    Wa = W1[0:4] + jnp.concatenate([Wr, pad], axis=0)
    Wb = W1[4:8] - jnp.concatenate([Wr, pad], axis=0)
    w1d = W1[10:11]
    c1x = W1[0] + W1[8]
    c1y = W1[1] + W1[9]
    c2x = W1[4] - W1[8]
    c2y = W1[5] - W1[9]

    eye4 = jnp.eye(_C, dtype=f32)
    ones1F = jnp.ones((1, _F), dtype=f32)

    def bcmat(vec):
        return jnp.kron(eye4, vec[:, None] @ ones1F)

    e0 = jnp.zeros((4, 1), dtype=f32).at[0, 0].set(1.0)
    e1 = jnp.zeros((4, 1), dtype=f32).at[1, 0].set(1.0)

    consts = (
        jnp.tile(Wa, (1, _C)),
        jnp.kron(eye4, Wb),
        jnp.kron(eye4, e0 @ ones1F),
        jnp.kron(eye4, e1 @ ones1F),
        jnp.tile(b1[None, :], (1, _C)),
        jnp.tile(w1d, (1, _C)),
        jnp.kron(eye4, W2),
        jnp.kron(eye4, W2.T),
        jnp.tile(b2[None, :], (1, _C)),
        jnp.tile(Wout[:, 0][None, :], (1, _C)),
        bcmat(w1d[0]),
        bcmat(c1x), bcmat(c1y), bcmat(c2x), bcmat(c2y),
        jnp.kron(eye4, jnp.ones((_F, 1), dtype=f32) / _F),
    )

    def body(_, xc):
        xr = xc.reshape(nb, _P, 4)
        xr4 = xc.reshape(nb, _Q, 16)
        gi, gj = _grad_step(xr, xr4, consts)
        gix = jnp.transpose(gi, (0, 2, 1, 3)).reshape(nb, 2, _P)
        gjx = gj[:, :, 0:4].reshape(nb, _P)
        gjy = gj[:, :, 4:8].reshape(nb, _P)
        gx = (gix[:, 0] + gjx).reshape(N)
        gy = (gix[:, 1] + gjy).reshape(N)
        newx = xc[:, 0:2] - 0.01 * jnp.stack([gx, gy], axis=1)
        return jnp.concatenate([newx, xc[:, 2:]], axis=1)

    return jax.lax.fori_loop(0, steps, body, x)
